# baseline (speedup 1.0000x reference)
import functools

import jax
import jax.numpy as jnp
from jax import lax
from jax.experimental import pallas as pl
from jax.experimental.pallas import tpu as pltpu
from jax.experimental.pallas import tpu_sc as plsc

NC = 2
NS = 16
NW = NC * NS


@functools.partial(jax.jit, static_argnames=("chunk",))
def _gather_rows(table, idx, chunk=1600):
    N = idx.shape[0]
    V, D = table.shape
    per_w = N // NW
    n_chunks = per_w // chunk
    assert per_w % chunk == 0 and N % NW == 0

    mesh = plsc.VectorSubcoreMesh(core_axis_name="c", subcore_axis_name="s")

    @functools.partial(
        pl.kernel,
        mesh=mesh,
        out_type=jax.ShapeDtypeStruct((N, D), jnp.float32),
        scratch_types=[
            pltpu.VMEM((chunk,), jnp.int32),
            pltpu.VMEM((chunk, D), jnp.float32),
            pltpu.SemaphoreType.DMA,
        ],
        compiler_params=pltpu.CompilerParams(use_tc_tiling_on_sc=False),
    )
    def k(table_hbm, idx_hbm, out_hbm, idx_v, rows_v, sem):
        wid = lax.axis_index("s") * NC + lax.axis_index("c")
        base = wid * per_w

        def body(i, carry):
            off = base + i * chunk
            pltpu.sync_copy(idx_hbm.at[pl.ds(off, chunk)], idx_v)
            pltpu.async_copy(table_hbm.at[idx_v], rows_v, sem).wait()
            pltpu.sync_copy(rows_v, out_hbm.at[pl.ds(off, chunk)])
            return carry

        lax.fori_loop(0, n_chunks, body, 0)

    return k(table, idx)


def kernel(x, table):
    B, H = x.shape
    D = table.shape[1]
    flat = x.reshape(B * H)
    out = _gather_rows(table, flat)
    return out.reshape(B, H, D)


# speedup vs baseline: 1.0063x; 1.0063x over previous
import functools

import jax
import jax.numpy as jnp
from jax import lax
from jax.experimental import pallas as pl
from jax.experimental.pallas import tpu as pltpu
from jax.experimental.pallas import tpu_sc as plsc

NC = 2
NS = 16
NW = NC * NS


@functools.partial(jax.jit, static_argnames=("chunk",))
def _gather_rows(table, idx, chunk=800):
    N = idx.shape[0]
    V, D = table.shape
    per_w = N // NW
    n_chunks = per_w // chunk
    assert per_w % chunk == 0 and N % NW == 0 and n_chunks % 2 == 0

    mesh = plsc.VectorSubcoreMesh(core_axis_name="c", subcore_axis_name="s")

    @functools.partial(
        pl.kernel,
        mesh=mesh,
        out_type=jax.ShapeDtypeStruct((N, D), jnp.float32),
        scratch_types=[
            pltpu.VMEM((per_w,), jnp.int32),
            pltpu.VMEM((2, chunk, D), jnp.float32),
            pltpu.SemaphoreType.DMA,
            pltpu.SemaphoreType.DMA,
            pltpu.SemaphoreType.DMA,
            pltpu.SemaphoreType.DMA,
        ],
        compiler_params=pltpu.CompilerParams(use_tc_tiling_on_sc=False),
    )
    def k(table_hbm, idx_hbm, out_hbm, idx_v, rows_v, g0, g1, w0, w1):
        wid = lax.axis_index("s") * NC + lax.axis_index("c")
        base = wid * per_w
        gsem = (g0, g1)
        wsem = (w0, w1)

        pltpu.sync_copy(idx_hbm.at[pl.ds(base, per_w)], idx_v)

        def gather(i, slot):
            return pltpu.async_copy(
                table_hbm.at[idx_v.at[pl.ds(i * chunk, chunk)]],
                rows_v.at[slot], gsem[slot])

        def writeback(i, slot):
            return pltpu.async_copy(
                rows_v.at[slot],
                out_hbm.at[pl.ds(base + i * chunk, chunk)], wsem[slot])

        cp = gather(0, 0)
        cp = gather(1, 1)

        def body(p, carry):
            for b in range(2):
                i = 2 * p + b
                pltpu.make_async_copy(
                    table_hbm.at[idx_v.at[pl.ds(i * chunk, chunk)]],
                    rows_v.at[b], gsem[b]).wait()
                writeback(i, b)

                @pl.when(i + 2 < n_chunks)
                def _():
                    pltpu.make_async_copy(
                        rows_v.at[b],
                        out_hbm.at[pl.ds(base + i * chunk, chunk)],
                        wsem[b]).wait()
                    gather(i + 2, b)
            return carry

        lax.fori_loop(0, n_chunks // 2, body, 0)

        for b in range(2):
            i = n_chunks - 2 + b
            pltpu.make_async_copy(
                rows_v.at[b],
                out_hbm.at[pl.ds(base + i * chunk, chunk)],
                wsem[b]).wait()

    return k(table, idx)


def kernel(x, table):
    B, H = x.shape
    D = table.shape[1]
    flat = x.reshape(B * H)
    out = _gather_rows(table, flat)
    return out.reshape(B, H, D)


# speedup vs baseline: 1.2318x; 1.2241x over previous
import functools

import jax
import jax.numpy as jnp
from jax import lax
from jax.experimental import pallas as pl
from jax.experimental.pallas import tpu as pltpu
from jax.experimental.pallas import tpu_sc as plsc

NC = 2
NS = 16
NW = NC * NS


@functools.partial(jax.jit, static_argnames=("chunk",))
def _gather_rows(tpad, idx, chunk=400):
    N = idx.shape[0]
    Vp, Dp = tpad.shape
    D = Dp // 2
    per_w = N // NW
    n_chunks = per_w // chunk
    assert per_w % chunk == 0 and N % NW == 0 and n_chunks % 2 == 0

    mesh = plsc.VectorSubcoreMesh(core_axis_name="c", subcore_axis_name="s")

    @functools.partial(
        pl.kernel,
        mesh=mesh,
        out_type=jax.ShapeDtypeStruct((N, Dp), jnp.float32),
        scratch_types=[
            pltpu.VMEM((per_w,), jnp.int32),
            pltpu.VMEM((2, chunk, Dp), jnp.float32),
            pltpu.SemaphoreType.DMA,
            pltpu.SemaphoreType.DMA,
            pltpu.SemaphoreType.DMA,
            pltpu.SemaphoreType.DMA,
        ],
        compiler_params=pltpu.CompilerParams(use_tc_tiling_on_sc=True),
    )
    def k(tpad_hbm, idx_hbm, out_hbm, idx_v, rows_v, g0, g1, w0, w1):
        wid = lax.axis_index("s") * NC + lax.axis_index("c")
        base = wid * per_w
        gsem = (g0, g1)
        wsem = (w0, w1)

        pltpu.sync_copy(idx_hbm.at[pl.ds(base, per_w)], idx_v)

        def gather(i, slot):
            return pltpu.async_copy(
                tpad_hbm.at[idx_v.at[pl.ds(i * chunk, chunk)]],
                rows_v.at[slot], gsem[slot])

        def writeback(i, slot):
            return pltpu.async_copy(
                rows_v.at[slot],
                out_hbm.at[pl.ds(base + i * chunk, chunk)], wsem[slot])

        gather(0, 0)
        gather(1, 1)

        def body(p, carry):
            for b in range(2):
                i = 2 * p + b
                pltpu.make_async_copy(
                    tpad_hbm.at[idx_v.at[pl.ds(i * chunk, chunk)]],
                    rows_v.at[b], gsem[b]).wait()
                writeback(i, b)

                @pl.when(i + 2 < n_chunks)
                def _():
                    pltpu.make_async_copy(
                        rows_v.at[b],
                        out_hbm.at[pl.ds(base + i * chunk, chunk)],
                        wsem[b]).wait()
                    gather(i + 2, b)
            return carry

        lax.fori_loop(0, n_chunks // 2, body, 0)

        for b in range(2):
            i = n_chunks - 2 + b
            pltpu.make_async_copy(
                rows_v.at[b],
                out_hbm.at[pl.ds(base + i * chunk, chunk)],
                wsem[b]).wait()

    return k(tpad, idx)


def kernel(x, table):
    B, H = x.shape
    V, D = table.shape
    vpad = (-V) % 8
    tpad = jnp.pad(table, ((0, vpad), (0, D)))
    flat = x.reshape(B * H)
    out = _gather_rows(tpad, flat)
    return out.reshape(B, H, 2 * D)[:, :, :D]
